# baseline (speedup 1.0000x reference)
import functools

import jax
import jax.numpy as jnp
from jax import lax
from jax.experimental import pallas as pl
from jax.experimental.pallas import tpu as pltpu
from jax.experimental.pallas import tpu_sc as plsc

N = 10000
D = 128
NP = 10240
E = 320000
EP = 327680
NC = 2
NS = 16
NW = NC * NS
EPW = EP // NW
CHUNK = 128
NCHUNK = EPW // CHUNK
RPT = NP // NS
RB = 1024
CHUNKS_C0 = 160
CHUNKS_C1 = 0

@functools.cache
def _sc_kernels():
    mesh = plsc.VectorSubcoreMesh(core_axis_name="c", subcore_axis_name="s",
                                  num_cores=NC, num_subcores=NS)

    @functools.partial(
        pl.kernel,
        out_type=jax.ShapeDtypeStruct((NW, NP), jnp.float32),
        mesh=mesh,
        scratch_types=[
            pltpu.VMEM((NCHUNK, CHUNK), jnp.int32),
            pltpu.VMEM((NP,), jnp.float32),
        ],
        compiler_params=pltpu.CompilerParams(needs_layout_passes=False),
    )
    def _sc_degree(dst_hbm, out_hbm, idx_v, hist):
        c = lax.axis_index("c")
        s = lax.axis_index("s")
        wid = s * NC + c

        def zero_body(i, _):
            hist[pl.ds(i * 16, 16)] = jnp.zeros((16,), jnp.float32)
            return 0

        lax.fori_loop(0, NP // 16, zero_body, 0)
        pltpu.sync_copy(dst_hbm.at[pl.ds(wid * NCHUNK, NCHUNK)], idx_v)
        ones = jnp.ones((16,), jnp.float32)

        def hist_body(i, _):
            for j in range(CHUNK // 16):
                idx = idx_v[i, pl.ds(j * 16, 16)]
                plsc.addupdate_scatter(hist, [idx], ones)
            return 0

        lax.fori_loop(0, NCHUNK, hist_body, 0)
        pltpu.sync_copy(hist, out_hbm.at[wid])

    STAGE = 40

    @functools.partial(
        pl.kernel,
        out_type=jax.ShapeDtypeStruct((NC * NP, D), jnp.float32),
        mesh=mesh,
        scratch_types=[
            pltpu.VMEM((STAGE, CHUNK), jnp.int32),
            pltpu.VMEM((STAGE, CHUNK), jnp.int32),
            pltpu.VMEM((CHUNK, D), jnp.float32),
            pltpu.VMEM((CHUNK, D), jnp.float32),
            pltpu.VMEM_SHARED((NP, D), jnp.float32),
            pltpu.SemaphoreType.DMA,
            pltpu.SemaphoreType.DMA,
        ],
    )
    def _sc_scatter(y_hbm, src_hbm, dst_hbm, zeros_hbm, out_hbm,
                    src_v, dst_v, rows_a, rows_b, acc, sem_a, sem_b):
        rows = (rows_a, rows_b)
        sems = (sem_a, sem_b)
        c = lax.axis_index("c")
        s = lax.axis_index("s")
        pltpu.sync_copy(zeros_hbm, acc.at[pl.ds(s * RPT, RPT)])
        plsc.subcore_barrier()

        def run_stage(base):
            pltpu.sync_copy(src_hbm.at[pl.ds(base, STAGE)], src_v)
            pltpu.sync_copy(dst_hbm.at[pl.ds(base, STAGE)], dst_v)
            gdesc = [pltpu.async_copy(y_hbm.at[src_v.at[0]], rows[0],
                                      sems[0]), None]
            for i in range(STAGE):
                b = i % 2
                if i + 1 < STAGE:
                    gdesc[1 - b] = pltpu.async_copy(
                        y_hbm.at[src_v.at[i + 1]], rows[1 - b], sems[1 - b])
                gdesc[b].wait()
                pltpu.sync_copy(rows[b], acc.at[dst_v.at[i]], add=True)

        nst0 = CHUNKS_C0 // STAGE
        nst1 = CHUNKS_C1 // STAGE

        @pl.when(c == 0)
        def _():
            for h in range(nst0):
                run_stage(s * CHUNKS_C0 + h * STAGE)

        @pl.when(c == 1)
        def _():
            for h in range(nst1):
                run_stage(NS * CHUNKS_C0 + s * CHUNKS_C1 + h * STAGE)

        plsc.subcore_barrier()
        pltpu.sync_copy(acc.at[pl.ds(s * RPT, RPT)],
                        out_hbm.at[pl.ds(c * NP + s * RPT, RPT)])

    return _sc_degree, _sc_scatter


def _dis_from_parts(degp):
    deg = jnp.sum(degp, axis=0) + 1.0
    return lax.rsqrt(deg)


def _tc_first_body(degp_ref, x_ref, w_ref, y_ref):
    dis = _dis_from_parts(degp_ref[...])
    xw = jnp.dot(x_ref[...], w_ref[...], preferred_element_type=jnp.float32)
    y_ref[...] = xw * dis[:, None]


_tc_first = pl.pallas_call(
    _tc_first_body,
    grid=(NP // RB,),
    in_specs=[
        pl.BlockSpec((NW, RB), lambda i: (0, i)),
        pl.BlockSpec((RB, D), lambda i: (i, 0)),
        pl.BlockSpec((D, D), lambda i: (0, 0)),
    ],
    out_specs=pl.BlockSpec((RB, D), lambda i: (i, 0)),
    out_shape=jax.ShapeDtypeStruct((NP, D), jnp.float32),
)


def _tc_mid_body(p_ref, y_ref, degp_ref, b_ref, w_ref, o_ref):
    dis = _dis_from_parts(degp_ref[...])
    agg = p_ref[0] + p_ref[1] + y_ref[...]
    h = jnp.maximum(dis[:, None] * agg + b_ref[...], 0.0)
    o_ref[...] = jnp.dot(h, w_ref[...],
                         preferred_element_type=jnp.float32) * dis[:, None]


_tc_mid = pl.pallas_call(
    _tc_mid_body,
    grid=(NP // RB,),
    in_specs=[
        pl.BlockSpec((2, RB, D), lambda i: (0, i, 0)),
        pl.BlockSpec((RB, D), lambda i: (i, 0)),
        pl.BlockSpec((NW, RB), lambda i: (0, i)),
        pl.BlockSpec((1, D), lambda i: (0, 0)),
        pl.BlockSpec((D, D), lambda i: (0, 0)),
    ],
    out_specs=pl.BlockSpec((RB, D), lambda i: (i, 0)),
    out_shape=jax.ShapeDtypeStruct((NP, D), jnp.float32),
)


def _tc_last_body(p_ref, y_ref, degp_ref, b_ref, wc_ref, bc_ref,
                  h_ref, o_ref):
    dis = _dis_from_parts(degp_ref[...])
    agg = p_ref[0] + p_ref[1] + y_ref[...]
    h = jnp.maximum(dis[:, None] * agg + b_ref[...], 0.0)
    h_ref[...] = h
    o_ref[...] = jnp.dot(h, wc_ref[...],
                         preferred_element_type=jnp.float32) + bc_ref[...]


_tc_last = pl.pallas_call(
    _tc_last_body,
    grid=(NP // RB,),
    in_specs=[
        pl.BlockSpec((2, RB, D), lambda i: (0, i, 0)),
        pl.BlockSpec((RB, D), lambda i: (i, 0)),
        pl.BlockSpec((NW, RB), lambda i: (0, i)),
        pl.BlockSpec((1, D), lambda i: (0, 0)),
        pl.BlockSpec((D, D), lambda i: (0, 0)),
        pl.BlockSpec((1, D), lambda i: (0, 0)),
    ],
    out_specs=[
        pl.BlockSpec((RB, D), lambda i: (i, 0)),
        pl.BlockSpec((RB, D), lambda i: (i, 0)),
    ],
    out_shape=[
        jax.ShapeDtypeStruct((NP, D), jnp.float32),
        jax.ShapeDtypeStruct((NP, D), jnp.float32),
    ],
)


def kernel(x, edge_index, W1, b1, W2, b2, W3, b3, Wc, bc):
    n_classes = Wc.shape[1]
    ei = edge_index.astype(jnp.int32)
    pad_e = EP - E
    pad_dst = N + (jnp.arange(pad_e, dtype=jnp.int32) % (NP - N))
    src = jnp.concatenate([ei[0], jnp.zeros((pad_e,), jnp.int32)])
    src = src.reshape(NW * NCHUNK, CHUNK)
    dst = jnp.concatenate([ei[1], pad_dst]).reshape(NW * NCHUNK, CHUNK)
    xp = jnp.zeros((NP, D), jnp.float32).at[:N].set(x)
    zeros_row = jnp.zeros((RPT, D), jnp.float32)
    wcp = jnp.zeros((D, D), jnp.float32).at[:, :n_classes].set(Wc)
    bcp = jnp.zeros((1, D), jnp.float32).at[0, :n_classes].set(bc)

    _sc_degree, _sc_scatter = _sc_kernels()
    degp = _sc_degree(dst)
    y1 = _tc_first(degp, xp, W1)
    p1 = _sc_scatter(y1, src, dst, zeros_row).reshape(NC, NP, D)
    y2 = _tc_mid(p1, y1, degp, b1.reshape(1, D), W2)
    p2 = _sc_scatter(y2, src, dst, zeros_row).reshape(NC, NP, D)
    y3 = _tc_mid(p2, y2, degp, b2.reshape(1, D), W3)
    p3 = _sc_scatter(y3, src, dst, zeros_row).reshape(NC, NP, D)
    h3, outp = _tc_last(p3, y3, degp, b3.reshape(1, D), wcp, bcp)
    return (outp[:N, :n_classes], h3[:N])


# speedup vs baseline: 1.1273x; 1.1273x over previous
import functools

import jax
import jax.numpy as jnp
from jax import lax
from jax.experimental import pallas as pl
from jax.experimental.pallas import tpu as pltpu
from jax.experimental.pallas import tpu_sc as plsc

N = 10000
D = 128
NP = 10240
E = 320000
EP = 327680
NC = 2
NS = 16
NW = NC * NS
EPW = EP // NW
CHUNK = 128
NCHUNK = EPW // CHUNK
RPT = NP // NS
RB = 1024
CHUNKS_C0 = 104
CHUNKS_C1 = 56

@functools.cache
def _sc_kernels():
    mesh = plsc.VectorSubcoreMesh(core_axis_name="c", subcore_axis_name="s",
                                  num_cores=NC, num_subcores=NS)

    @functools.partial(
        pl.kernel,
        out_type=jax.ShapeDtypeStruct((NW, NP), jnp.float32),
        mesh=mesh,
        scratch_types=[
            pltpu.VMEM((NCHUNK, CHUNK), jnp.int32),
            pltpu.VMEM((NP,), jnp.float32),
        ],
        compiler_params=pltpu.CompilerParams(needs_layout_passes=False),
    )
    def _sc_degree(dst_hbm, out_hbm, idx_v, hist):
        c = lax.axis_index("c")
        s = lax.axis_index("s")
        wid = s * NC + c

        def zero_body(i, _):
            hist[pl.ds(i * 16, 16)] = jnp.zeros((16,), jnp.float32)
            return 0

        lax.fori_loop(0, NP // 16, zero_body, 0)
        pltpu.sync_copy(dst_hbm.at[pl.ds(wid * NCHUNK, NCHUNK)], idx_v)
        ones = jnp.ones((16,), jnp.float32)

        def hist_body(i, _):
            for j in range(CHUNK // 16):
                idx = idx_v[i, pl.ds(j * 16, 16)]
                plsc.addupdate_scatter(hist, [idx], ones)
            return 0

        lax.fori_loop(0, NCHUNK, hist_body, 0)
        pltpu.sync_copy(hist, out_hbm.at[wid])

    STAGE = 8

    @functools.partial(
        pl.kernel,
        out_type=jax.ShapeDtypeStruct((NC * NP, D), jnp.float32),
        mesh=mesh,
        scratch_types=[
            pltpu.VMEM((STAGE, CHUNK), jnp.int32),
            pltpu.VMEM((STAGE, CHUNK), jnp.int32),
            pltpu.VMEM((CHUNK, D), jnp.float32),
            pltpu.VMEM((CHUNK, D), jnp.float32),
            pltpu.VMEM_SHARED((NP, D), jnp.float32),
            pltpu.SemaphoreType.DMA,
            pltpu.SemaphoreType.DMA,
        ],
    )
    def _sc_scatter(y_hbm, src_hbm, dst_hbm, zeros_hbm, out_hbm,
                    src_v, dst_v, rows_a, rows_b, acc, sem_a, sem_b):
        rows = (rows_a, rows_b)
        sems = (sem_a, sem_b)
        c = lax.axis_index("c")
        s = lax.axis_index("s")
        pltpu.sync_copy(zeros_hbm, acc.at[pl.ds(s * RPT, RPT)])
        plsc.subcore_barrier()

        def run_stage(base):
            pltpu.sync_copy(src_hbm.at[pl.ds(base, STAGE)], src_v)
            pltpu.sync_copy(dst_hbm.at[pl.ds(base, STAGE)], dst_v)
            gdesc = [pltpu.async_copy(y_hbm.at[src_v.at[0]], rows[0],
                                      sems[0]), None]
            for i in range(STAGE):
                b = i % 2
                if i + 1 < STAGE:
                    gdesc[1 - b] = pltpu.async_copy(
                        y_hbm.at[src_v.at[i + 1]], rows[1 - b], sems[1 - b])
                gdesc[b].wait()
                pltpu.sync_copy(rows[b], acc.at[dst_v.at[i]], add=True)

        nst0 = CHUNKS_C0 // STAGE
        nst1 = CHUNKS_C1 // STAGE

        @pl.when(c == 0)
        def _():
            for h in range(nst0):
                run_stage(s * CHUNKS_C0 + h * STAGE)

        @pl.when(c == 1)
        def _():
            for h in range(nst1):
                run_stage(NS * CHUNKS_C0 + s * CHUNKS_C1 + h * STAGE)

        plsc.subcore_barrier()
        pltpu.sync_copy(acc.at[pl.ds(s * RPT, RPT)],
                        out_hbm.at[pl.ds(c * NP + s * RPT, RPT)])

    return _sc_degree, _sc_scatter


def _dis_from_parts(degp):
    deg = jnp.sum(degp, axis=0) + 1.0
    return lax.rsqrt(deg)


def _tc_first_body(degp_ref, x_ref, w_ref, y_ref):
    dis = _dis_from_parts(degp_ref[...])
    xw = jnp.dot(x_ref[...], w_ref[...], preferred_element_type=jnp.float32)
    y_ref[...] = xw * dis[:, None]


_tc_first = pl.pallas_call(
    _tc_first_body,
    grid=(NP // RB,),
    in_specs=[
        pl.BlockSpec((NW, RB), lambda i: (0, i)),
        pl.BlockSpec((RB, D), lambda i: (i, 0)),
        pl.BlockSpec((D, D), lambda i: (0, 0)),
    ],
    out_specs=pl.BlockSpec((RB, D), lambda i: (i, 0)),
    out_shape=jax.ShapeDtypeStruct((NP, D), jnp.float32),
)


def _tc_mid_body(p_ref, y_ref, degp_ref, b_ref, w_ref, o_ref):
    dis = _dis_from_parts(degp_ref[...])
    agg = p_ref[0] + p_ref[1] + y_ref[...]
    h = jnp.maximum(dis[:, None] * agg + b_ref[...], 0.0)
    o_ref[...] = jnp.dot(h, w_ref[...],
                         preferred_element_type=jnp.float32) * dis[:, None]


_tc_mid = pl.pallas_call(
    _tc_mid_body,
    grid=(NP // RB,),
    in_specs=[
        pl.BlockSpec((2, RB, D), lambda i: (0, i, 0)),
        pl.BlockSpec((RB, D), lambda i: (i, 0)),
        pl.BlockSpec((NW, RB), lambda i: (0, i)),
        pl.BlockSpec((1, D), lambda i: (0, 0)),
        pl.BlockSpec((D, D), lambda i: (0, 0)),
    ],
    out_specs=pl.BlockSpec((RB, D), lambda i: (i, 0)),
    out_shape=jax.ShapeDtypeStruct((NP, D), jnp.float32),
)


def _tc_last_body(p_ref, y_ref, degp_ref, b_ref, wc_ref, bc_ref,
                  h_ref, o_ref):
    dis = _dis_from_parts(degp_ref[...])
    agg = p_ref[0] + p_ref[1] + y_ref[...]
    h = jnp.maximum(dis[:, None] * agg + b_ref[...], 0.0)
    h_ref[...] = h
    o_ref[...] = jnp.dot(h, wc_ref[...],
                         preferred_element_type=jnp.float32) + bc_ref[...]


_tc_last = pl.pallas_call(
    _tc_last_body,
    grid=(NP // RB,),
    in_specs=[
        pl.BlockSpec((2, RB, D), lambda i: (0, i, 0)),
        pl.BlockSpec((RB, D), lambda i: (i, 0)),
        pl.BlockSpec((NW, RB), lambda i: (0, i)),
        pl.BlockSpec((1, D), lambda i: (0, 0)),
        pl.BlockSpec((D, D), lambda i: (0, 0)),
        pl.BlockSpec((1, D), lambda i: (0, 0)),
    ],
    out_specs=[
        pl.BlockSpec((RB, D), lambda i: (i, 0)),
        pl.BlockSpec((RB, D), lambda i: (i, 0)),
    ],
    out_shape=[
        jax.ShapeDtypeStruct((NP, D), jnp.float32),
        jax.ShapeDtypeStruct((NP, D), jnp.float32),
    ],
)


def kernel(x, edge_index, W1, b1, W2, b2, W3, b3, Wc, bc):
    n_classes = Wc.shape[1]
    ei = edge_index.astype(jnp.int32)
    pad_e = EP - E
    pad_dst = N + (jnp.arange(pad_e, dtype=jnp.int32) % (NP - N))
    src = jnp.concatenate([ei[0], jnp.zeros((pad_e,), jnp.int32)])
    src = src.reshape(NW * NCHUNK, CHUNK)
    dst = jnp.concatenate([ei[1], pad_dst]).reshape(NW * NCHUNK, CHUNK)
    xp = jnp.zeros((NP, D), jnp.float32).at[:N].set(x)
    zeros_row = jnp.zeros((RPT, D), jnp.float32)
    wcp = jnp.zeros((D, D), jnp.float32).at[:, :n_classes].set(Wc)
    bcp = jnp.zeros((1, D), jnp.float32).at[0, :n_classes].set(bc)

    _sc_degree, _sc_scatter = _sc_kernels()
    degp = _sc_degree(dst)
    y1 = _tc_first(degp, xp, W1)
    p1 = _sc_scatter(y1, src, dst, zeros_row).reshape(NC, NP, D)
    y2 = _tc_mid(p1, y1, degp, b1.reshape(1, D), W2)
    p2 = _sc_scatter(y2, src, dst, zeros_row).reshape(NC, NP, D)
    y3 = _tc_mid(p2, y2, degp, b2.reshape(1, D), W3)
    p3 = _sc_scatter(y3, src, dst, zeros_row).reshape(NC, NP, D)
    h3, outp = _tc_last(p3, y3, degp, b3.reshape(1, D), wcp, bcp)
    return (outp[:N, :n_classes], h3[:N])


# speedup vs baseline: 1.1755x; 1.0427x over previous
import functools

import jax
import jax.numpy as jnp
from jax import lax
from jax.experimental import pallas as pl
from jax.experimental.pallas import tpu as pltpu
from jax.experimental.pallas import tpu_sc as plsc

N = 10000
D = 128
NP = 10240
E = 320000
EP = 327680
NC = 2
NS = 16
NW = NC * NS
EPW = EP // NW
CHUNK = 128
NCHUNK = EPW // CHUNK
RPT = NP // NS
RB = 1024
CHUNKS_C0 = 120
CHUNKS_C1 = 40

@functools.cache
def _sc_kernels():
    mesh = plsc.VectorSubcoreMesh(core_axis_name="c", subcore_axis_name="s",
                                  num_cores=NC, num_subcores=NS)

    @functools.partial(
        pl.kernel,
        out_type=jax.ShapeDtypeStruct((NW, NP), jnp.float32),
        mesh=mesh,
        scratch_types=[
            pltpu.VMEM((NCHUNK, CHUNK), jnp.int32),
            pltpu.VMEM((NP,), jnp.float32),
        ],
        compiler_params=pltpu.CompilerParams(needs_layout_passes=False),
    )
    def _sc_degree(dst_hbm, out_hbm, idx_v, hist):
        c = lax.axis_index("c")
        s = lax.axis_index("s")
        wid = s * NC + c

        def zero_body(i, _):
            hist[pl.ds(i * 16, 16)] = jnp.zeros((16,), jnp.float32)
            return 0

        lax.fori_loop(0, NP // 16, zero_body, 0)
        pltpu.sync_copy(dst_hbm.at[pl.ds(wid * NCHUNK, NCHUNK)], idx_v)
        ones = jnp.ones((16,), jnp.float32)

        def hist_body(i, _):
            for j in range(CHUNK // 16):
                idx = idx_v[i, pl.ds(j * 16, 16)]
                plsc.addupdate_scatter(hist, [idx], ones)
            return 0

        lax.fori_loop(0, NCHUNK, hist_body, 0)
        pltpu.sync_copy(hist, out_hbm.at[wid])

    STAGE = 8

    @functools.partial(
        pl.kernel,
        out_type=jax.ShapeDtypeStruct((NC * NP, D), jnp.float32),
        mesh=mesh,
        scratch_types=[
            pltpu.VMEM((STAGE, CHUNK), jnp.int32),
            pltpu.VMEM((STAGE, CHUNK), jnp.int32),
            pltpu.VMEM((CHUNK, D), jnp.float32),
            pltpu.VMEM((CHUNK, D), jnp.float32),
            pltpu.VMEM_SHARED((NP, D), jnp.float32),
            pltpu.SemaphoreType.DMA,
            pltpu.SemaphoreType.DMA,
        ],
    )
    def _sc_scatter(y_hbm, src_hbm, dst_hbm, zeros_hbm, out_hbm,
                    src_v, dst_v, rows_a, rows_b, acc, sem_a, sem_b):
        rows = (rows_a, rows_b)
        sems = (sem_a, sem_b)
        c = lax.axis_index("c")
        s = lax.axis_index("s")
        pltpu.sync_copy(zeros_hbm, acc.at[pl.ds(s * RPT, RPT)])
        plsc.subcore_barrier()

        def run_stage(base):
            pltpu.sync_copy(src_hbm.at[pl.ds(base, STAGE)], src_v)
            pltpu.sync_copy(dst_hbm.at[pl.ds(base, STAGE)], dst_v)
            gdesc = [pltpu.async_copy(y_hbm.at[src_v.at[0]], rows[0],
                                      sems[0]), None]
            for i in range(STAGE):
                b = i % 2
                if i + 1 < STAGE:
                    gdesc[1 - b] = pltpu.async_copy(
                        y_hbm.at[src_v.at[i + 1]], rows[1 - b], sems[1 - b])
                gdesc[b].wait()
                pltpu.sync_copy(rows[b], acc.at[dst_v.at[i]], add=True)

        nst0 = CHUNKS_C0 // STAGE
        nst1 = CHUNKS_C1 // STAGE

        @pl.when(c == 0)
        def _():
            for h in range(nst0):
                run_stage(s * CHUNKS_C0 + h * STAGE)

        @pl.when(c == 1)
        def _():
            for h in range(nst1):
                run_stage(NS * CHUNKS_C0 + s * CHUNKS_C1 + h * STAGE)

        plsc.subcore_barrier()
        pltpu.sync_copy(acc.at[pl.ds(s * RPT, RPT)],
                        out_hbm.at[pl.ds(c * NP + s * RPT, RPT)])

    return _sc_degree, _sc_scatter


def _dis_from_parts(degp):
    deg = jnp.sum(degp, axis=0) + 1.0
    return lax.rsqrt(deg)


def _tc_first_body(degp_ref, x_ref, w_ref, y_ref):
    dis = _dis_from_parts(degp_ref[...])
    xw = jnp.dot(x_ref[...], w_ref[...], preferred_element_type=jnp.float32)
    y_ref[...] = xw * dis[:, None]


_tc_first = pl.pallas_call(
    _tc_first_body,
    grid=(NP // RB,),
    in_specs=[
        pl.BlockSpec((NW, RB), lambda i: (0, i)),
        pl.BlockSpec((RB, D), lambda i: (i, 0)),
        pl.BlockSpec((D, D), lambda i: (0, 0)),
    ],
    out_specs=pl.BlockSpec((RB, D), lambda i: (i, 0)),
    out_shape=jax.ShapeDtypeStruct((NP, D), jnp.float32),
)


def _tc_mid_body(p_ref, y_ref, degp_ref, b_ref, w_ref, o_ref):
    dis = _dis_from_parts(degp_ref[...])
    agg = p_ref[0] + p_ref[1] + y_ref[...]
    h = jnp.maximum(dis[:, None] * agg + b_ref[...], 0.0)
    o_ref[...] = jnp.dot(h, w_ref[...],
                         preferred_element_type=jnp.float32) * dis[:, None]


_tc_mid = pl.pallas_call(
    _tc_mid_body,
    grid=(NP // RB,),
    in_specs=[
        pl.BlockSpec((2, RB, D), lambda i: (0, i, 0)),
        pl.BlockSpec((RB, D), lambda i: (i, 0)),
        pl.BlockSpec((NW, RB), lambda i: (0, i)),
        pl.BlockSpec((1, D), lambda i: (0, 0)),
        pl.BlockSpec((D, D), lambda i: (0, 0)),
    ],
    out_specs=pl.BlockSpec((RB, D), lambda i: (i, 0)),
    out_shape=jax.ShapeDtypeStruct((NP, D), jnp.float32),
)


def _tc_last_body(p_ref, y_ref, degp_ref, b_ref, wc_ref, bc_ref,
                  h_ref, o_ref):
    dis = _dis_from_parts(degp_ref[...])
    agg = p_ref[0] + p_ref[1] + y_ref[...]
    h = jnp.maximum(dis[:, None] * agg + b_ref[...], 0.0)
    h_ref[...] = h
    o_ref[...] = jnp.dot(h, wc_ref[...],
                         preferred_element_type=jnp.float32) + bc_ref[...]


_tc_last = pl.pallas_call(
    _tc_last_body,
    grid=(NP // RB,),
    in_specs=[
        pl.BlockSpec((2, RB, D), lambda i: (0, i, 0)),
        pl.BlockSpec((RB, D), lambda i: (i, 0)),
        pl.BlockSpec((NW, RB), lambda i: (0, i)),
        pl.BlockSpec((1, D), lambda i: (0, 0)),
        pl.BlockSpec((D, D), lambda i: (0, 0)),
        pl.BlockSpec((1, D), lambda i: (0, 0)),
    ],
    out_specs=[
        pl.BlockSpec((RB, D), lambda i: (i, 0)),
        pl.BlockSpec((RB, D), lambda i: (i, 0)),
    ],
    out_shape=[
        jax.ShapeDtypeStruct((NP, D), jnp.float32),
        jax.ShapeDtypeStruct((NP, D), jnp.float32),
    ],
)


def kernel(x, edge_index, W1, b1, W2, b2, W3, b3, Wc, bc):
    n_classes = Wc.shape[1]
    ei = edge_index.astype(jnp.int32)
    pad_e = EP - E
    pad_dst = N + (jnp.arange(pad_e, dtype=jnp.int32) % (NP - N))
    src = jnp.concatenate([ei[0], jnp.zeros((pad_e,), jnp.int32)])
    src = src.reshape(NW * NCHUNK, CHUNK)
    dst = jnp.concatenate([ei[1], pad_dst]).reshape(NW * NCHUNK, CHUNK)
    xp = jnp.zeros((NP, D), jnp.float32).at[:N].set(x)
    zeros_row = jnp.zeros((RPT, D), jnp.float32)
    wcp = jnp.zeros((D, D), jnp.float32).at[:, :n_classes].set(Wc)
    bcp = jnp.zeros((1, D), jnp.float32).at[0, :n_classes].set(bc)

    _sc_degree, _sc_scatter = _sc_kernels()
    degp = _sc_degree(dst)
    y1 = _tc_first(degp, xp, W1)
    p1 = _sc_scatter(y1, src, dst, zeros_row).reshape(NC, NP, D)
    y2 = _tc_mid(p1, y1, degp, b1.reshape(1, D), W2)
    p2 = _sc_scatter(y2, src, dst, zeros_row).reshape(NC, NP, D)
    y3 = _tc_mid(p2, y2, degp, b2.reshape(1, D), W3)
    p3 = _sc_scatter(y3, src, dst, zeros_row).reshape(NC, NP, D)
    h3, outp = _tc_last(p3, y3, degp, b3.reshape(1, D), wcp, bcp)
    return (outp[:N, :n_classes], h3[:N])


# speedup vs baseline: 1.1997x; 1.0206x over previous
import functools

import jax
import jax.numpy as jnp
from jax import lax
from jax.experimental import pallas as pl
from jax.experimental.pallas import tpu as pltpu
from jax.experimental.pallas import tpu_sc as plsc

N = 10000
D = 128
NP = 10240
E = 320000
EP = 327680
NC = 2
NS = 16
NW = NC * NS
EPW = EP // NW
CHUNK = 128
NCHUNK = EPW // CHUNK
RPT = NP // NS
RB = 1024
CHUNKS_C0 = 128
CHUNKS_C1 = 32

@functools.cache
def _sc_kernels():
    mesh = plsc.VectorSubcoreMesh(core_axis_name="c", subcore_axis_name="s",
                                  num_cores=NC, num_subcores=NS)

    @functools.partial(
        pl.kernel,
        out_type=jax.ShapeDtypeStruct((NW, NP), jnp.float32),
        mesh=mesh,
        scratch_types=[
            pltpu.VMEM((NCHUNK, CHUNK), jnp.int32),
            pltpu.VMEM((NP,), jnp.float32),
        ],
        compiler_params=pltpu.CompilerParams(needs_layout_passes=False),
    )
    def _sc_degree(dst_hbm, out_hbm, idx_v, hist):
        c = lax.axis_index("c")
        s = lax.axis_index("s")
        wid = s * NC + c

        def zero_body(i, _):
            hist[pl.ds(i * 16, 16)] = jnp.zeros((16,), jnp.float32)
            return 0

        lax.fori_loop(0, NP // 16, zero_body, 0)
        pltpu.sync_copy(dst_hbm.at[pl.ds(wid * NCHUNK, NCHUNK)], idx_v)
        ones = jnp.ones((16,), jnp.float32)

        def hist_body(i, _):
            for j in range(CHUNK // 16):
                idx = idx_v[i, pl.ds(j * 16, 16)]
                plsc.addupdate_scatter(hist, [idx], ones)
            return 0

        lax.fori_loop(0, NCHUNK, hist_body, 0)
        pltpu.sync_copy(hist, out_hbm.at[wid])

    STAGE = 8

    @functools.partial(
        pl.kernel,
        out_type=jax.ShapeDtypeStruct((NC * NP, D), jnp.float32),
        mesh=mesh,
        scratch_types=[
            pltpu.VMEM((STAGE, CHUNK), jnp.int32),
            pltpu.VMEM((STAGE, CHUNK), jnp.int32),
            pltpu.VMEM((CHUNK, D), jnp.float32),
            pltpu.VMEM((CHUNK, D), jnp.float32),
            pltpu.VMEM_SHARED((NP, D), jnp.float32),
            pltpu.SemaphoreType.DMA,
            pltpu.SemaphoreType.DMA,
        ],
    )
    def _sc_scatter(y_hbm, src_hbm, dst_hbm, zeros_hbm, out_hbm,
                    src_v, dst_v, rows_a, rows_b, acc, sem_a, sem_b):
        rows = (rows_a, rows_b)
        sems = (sem_a, sem_b)
        c = lax.axis_index("c")
        s = lax.axis_index("s")
        pltpu.sync_copy(zeros_hbm, acc.at[pl.ds(s * RPT, RPT)])
        plsc.subcore_barrier()

        def run_stage(base):
            pltpu.sync_copy(src_hbm.at[pl.ds(base, STAGE)], src_v)
            pltpu.sync_copy(dst_hbm.at[pl.ds(base, STAGE)], dst_v)
            gdesc = [pltpu.async_copy(y_hbm.at[src_v.at[0]], rows[0],
                                      sems[0]), None]
            for i in range(STAGE):
                b = i % 2
                if i + 1 < STAGE:
                    gdesc[1 - b] = pltpu.async_copy(
                        y_hbm.at[src_v.at[i + 1]], rows[1 - b], sems[1 - b])
                gdesc[b].wait()
                pltpu.sync_copy(rows[b], acc.at[dst_v.at[i]], add=True)

        nst0 = CHUNKS_C0 // STAGE
        nst1 = CHUNKS_C1 // STAGE

        @pl.when(c == 0)
        def _():
            for h in range(nst0):
                run_stage(s * CHUNKS_C0 + h * STAGE)

        @pl.when(c == 1)
        def _():
            for h in range(nst1):
                run_stage(NS * CHUNKS_C0 + s * CHUNKS_C1 + h * STAGE)

        plsc.subcore_barrier()
        pltpu.sync_copy(acc.at[pl.ds(s * RPT, RPT)],
                        out_hbm.at[pl.ds(c * NP + s * RPT, RPT)])

    return _sc_degree, _sc_scatter


def _dis_from_parts(degp):
    deg = jnp.sum(degp, axis=0) + 1.0
    return lax.rsqrt(deg)


def _tc_first_body(degp_ref, x_ref, w_ref, y_ref):
    dis = _dis_from_parts(degp_ref[...])
    xw = jnp.dot(x_ref[...], w_ref[...], preferred_element_type=jnp.float32)
    y_ref[...] = xw * dis[:, None]


_tc_first = pl.pallas_call(
    _tc_first_body,
    grid=(NP // RB,),
    in_specs=[
        pl.BlockSpec((NW, RB), lambda i: (0, i)),
        pl.BlockSpec((RB, D), lambda i: (i, 0)),
        pl.BlockSpec((D, D), lambda i: (0, 0)),
    ],
    out_specs=pl.BlockSpec((RB, D), lambda i: (i, 0)),
    out_shape=jax.ShapeDtypeStruct((NP, D), jnp.float32),
)


def _tc_mid_body(p_ref, y_ref, degp_ref, b_ref, w_ref, o_ref):
    dis = _dis_from_parts(degp_ref[...])
    agg = p_ref[0] + p_ref[1] + y_ref[...]
    h = jnp.maximum(dis[:, None] * agg + b_ref[...], 0.0)
    o_ref[...] = jnp.dot(h, w_ref[...],
                         preferred_element_type=jnp.float32) * dis[:, None]


_tc_mid = pl.pallas_call(
    _tc_mid_body,
    grid=(NP // RB,),
    in_specs=[
        pl.BlockSpec((2, RB, D), lambda i: (0, i, 0)),
        pl.BlockSpec((RB, D), lambda i: (i, 0)),
        pl.BlockSpec((NW, RB), lambda i: (0, i)),
        pl.BlockSpec((1, D), lambda i: (0, 0)),
        pl.BlockSpec((D, D), lambda i: (0, 0)),
    ],
    out_specs=pl.BlockSpec((RB, D), lambda i: (i, 0)),
    out_shape=jax.ShapeDtypeStruct((NP, D), jnp.float32),
)


def _tc_last_body(p_ref, y_ref, degp_ref, b_ref, wc_ref, bc_ref,
                  h_ref, o_ref):
    dis = _dis_from_parts(degp_ref[...])
    agg = p_ref[0] + p_ref[1] + y_ref[...]
    h = jnp.maximum(dis[:, None] * agg + b_ref[...], 0.0)
    h_ref[...] = h
    o_ref[...] = jnp.dot(h, wc_ref[...],
                         preferred_element_type=jnp.float32) + bc_ref[...]


_tc_last = pl.pallas_call(
    _tc_last_body,
    grid=(NP // RB,),
    in_specs=[
        pl.BlockSpec((2, RB, D), lambda i: (0, i, 0)),
        pl.BlockSpec((RB, D), lambda i: (i, 0)),
        pl.BlockSpec((NW, RB), lambda i: (0, i)),
        pl.BlockSpec((1, D), lambda i: (0, 0)),
        pl.BlockSpec((D, D), lambda i: (0, 0)),
        pl.BlockSpec((1, D), lambda i: (0, 0)),
    ],
    out_specs=[
        pl.BlockSpec((RB, D), lambda i: (i, 0)),
        pl.BlockSpec((RB, D), lambda i: (i, 0)),
    ],
    out_shape=[
        jax.ShapeDtypeStruct((NP, D), jnp.float32),
        jax.ShapeDtypeStruct((NP, D), jnp.float32),
    ],
)


def kernel(x, edge_index, W1, b1, W2, b2, W3, b3, Wc, bc):
    n_classes = Wc.shape[1]
    ei = edge_index.astype(jnp.int32)
    pad_e = EP - E
    pad_dst = N + (jnp.arange(pad_e, dtype=jnp.int32) % (NP - N))
    src = jnp.concatenate([ei[0], jnp.zeros((pad_e,), jnp.int32)])
    src = src.reshape(NW * NCHUNK, CHUNK)
    dst = jnp.concatenate([ei[1], pad_dst]).reshape(NW * NCHUNK, CHUNK)
    xp = jnp.zeros((NP, D), jnp.float32).at[:N].set(x)
    zeros_row = jnp.zeros((RPT, D), jnp.float32)
    wcp = jnp.zeros((D, D), jnp.float32).at[:, :n_classes].set(Wc)
    bcp = jnp.zeros((1, D), jnp.float32).at[0, :n_classes].set(bc)

    _sc_degree, _sc_scatter = _sc_kernels()
    degp = _sc_degree(dst)
    y1 = _tc_first(degp, xp, W1)
    p1 = _sc_scatter(y1, src, dst, zeros_row).reshape(NC, NP, D)
    y2 = _tc_mid(p1, y1, degp, b1.reshape(1, D), W2)
    p2 = _sc_scatter(y2, src, dst, zeros_row).reshape(NC, NP, D)
    y3 = _tc_mid(p2, y2, degp, b2.reshape(1, D), W3)
    p3 = _sc_scatter(y3, src, dst, zeros_row).reshape(NC, NP, D)
    h3, outp = _tc_last(p3, y3, degp, b3.reshape(1, D), wcp, bcp)
    return (outp[:N, :n_classes], h3[:N])


# speedup vs baseline: 1.2361x; 1.0303x over previous
import functools

import jax
import jax.numpy as jnp
from jax import lax
from jax.experimental import pallas as pl
from jax.experimental.pallas import tpu as pltpu
from jax.experimental.pallas import tpu_sc as plsc

N = 10000
D = 128
NP = 10240
E = 320000
EP = 327680
NC = 2
NS = 16
NW = NC * NS
EPW = EP // NW
CHUNK = 128
NCHUNK = EPW // CHUNK
RPT = NP // NS
RB = 1024
CHUNKS_C0 = 136
CHUNKS_C1 = 24

@functools.cache
def _sc_kernels():
    mesh = plsc.VectorSubcoreMesh(core_axis_name="c", subcore_axis_name="s",
                                  num_cores=NC, num_subcores=NS)

    @functools.partial(
        pl.kernel,
        out_type=jax.ShapeDtypeStruct((NW, NP), jnp.float32),
        mesh=mesh,
        scratch_types=[
            pltpu.VMEM((NCHUNK, CHUNK), jnp.int32),
            pltpu.VMEM((NP,), jnp.float32),
        ],
        compiler_params=pltpu.CompilerParams(needs_layout_passes=False),
    )
    def _sc_degree(dst_hbm, out_hbm, idx_v, hist):
        c = lax.axis_index("c")
        s = lax.axis_index("s")
        wid = s * NC + c

        def zero_body(i, _):
            hist[pl.ds(i * 16, 16)] = jnp.zeros((16,), jnp.float32)
            return 0

        lax.fori_loop(0, NP // 16, zero_body, 0)
        pltpu.sync_copy(dst_hbm.at[pl.ds(wid * NCHUNK, NCHUNK)], idx_v)
        ones = jnp.ones((16,), jnp.float32)

        def hist_body(i, _):
            for j in range(CHUNK // 16):
                idx = idx_v[i, pl.ds(j * 16, 16)]
                plsc.addupdate_scatter(hist, [idx], ones)
            return 0

        lax.fori_loop(0, NCHUNK, hist_body, 0)
        pltpu.sync_copy(hist, out_hbm.at[wid])

    STAGE = 8

    @functools.partial(
        pl.kernel,
        out_type=jax.ShapeDtypeStruct((NC * NP, D), jnp.float32),
        mesh=mesh,
        scratch_types=[
            pltpu.VMEM((STAGE, CHUNK), jnp.int32),
            pltpu.VMEM((STAGE, CHUNK), jnp.int32),
            pltpu.VMEM((CHUNK, D), jnp.float32),
            pltpu.VMEM((CHUNK, D), jnp.float32),
            pltpu.VMEM_SHARED((NP, D), jnp.float32),
            pltpu.SemaphoreType.DMA,
            pltpu.SemaphoreType.DMA,
        ],
    )
    def _sc_scatter(y_hbm, src_hbm, dst_hbm, zeros_hbm, out_hbm,
                    src_v, dst_v, rows_a, rows_b, acc, sem_a, sem_b):
        rows = (rows_a, rows_b)
        sems = (sem_a, sem_b)
        c = lax.axis_index("c")
        s = lax.axis_index("s")
        pltpu.sync_copy(zeros_hbm, acc.at[pl.ds(s * RPT, RPT)])
        plsc.subcore_barrier()

        def run_stage(base):
            pltpu.sync_copy(src_hbm.at[pl.ds(base, STAGE)], src_v)
            pltpu.sync_copy(dst_hbm.at[pl.ds(base, STAGE)], dst_v)
            gdesc = [pltpu.async_copy(y_hbm.at[src_v.at[0]], rows[0],
                                      sems[0]), None]
            for i in range(STAGE):
                b = i % 2
                if i + 1 < STAGE:
                    gdesc[1 - b] = pltpu.async_copy(
                        y_hbm.at[src_v.at[i + 1]], rows[1 - b], sems[1 - b])
                gdesc[b].wait()
                pltpu.sync_copy(rows[b], acc.at[dst_v.at[i]], add=True)

        nst0 = CHUNKS_C0 // STAGE
        nst1 = CHUNKS_C1 // STAGE

        @pl.when(c == 0)
        def _():
            for h in range(nst0):
                run_stage(s * CHUNKS_C0 + h * STAGE)

        @pl.when(c == 1)
        def _():
            for h in range(nst1):
                run_stage(NS * CHUNKS_C0 + s * CHUNKS_C1 + h * STAGE)

        plsc.subcore_barrier()
        pltpu.sync_copy(acc.at[pl.ds(s * RPT, RPT)],
                        out_hbm.at[pl.ds(c * NP + s * RPT, RPT)])

    return _sc_degree, _sc_scatter


def _dis_from_parts(degp):
    deg = jnp.sum(degp, axis=0) + 1.0
    return lax.rsqrt(deg)


def _tc_first_body(degp_ref, x_ref, w_ref, y_ref):
    dis = _dis_from_parts(degp_ref[...])
    xw = jnp.dot(x_ref[...], w_ref[...], preferred_element_type=jnp.float32)
    y_ref[...] = xw * dis[:, None]


_tc_first = pl.pallas_call(
    _tc_first_body,
    grid=(NP // RB,),
    in_specs=[
        pl.BlockSpec((NW, RB), lambda i: (0, i)),
        pl.BlockSpec((RB, D), lambda i: (i, 0)),
        pl.BlockSpec((D, D), lambda i: (0, 0)),
    ],
    out_specs=pl.BlockSpec((RB, D), lambda i: (i, 0)),
    out_shape=jax.ShapeDtypeStruct((NP, D), jnp.float32),
)


def _tc_mid_body(p_ref, y_ref, degp_ref, b_ref, w_ref, o_ref):
    dis = _dis_from_parts(degp_ref[...])
    agg = p_ref[0] + p_ref[1] + y_ref[...]
    h = jnp.maximum(dis[:, None] * agg + b_ref[...], 0.0)
    o_ref[...] = jnp.dot(h, w_ref[...],
                         preferred_element_type=jnp.float32) * dis[:, None]


_tc_mid = pl.pallas_call(
    _tc_mid_body,
    grid=(NP // RB,),
    in_specs=[
        pl.BlockSpec((2, RB, D), lambda i: (0, i, 0)),
        pl.BlockSpec((RB, D), lambda i: (i, 0)),
        pl.BlockSpec((NW, RB), lambda i: (0, i)),
        pl.BlockSpec((1, D), lambda i: (0, 0)),
        pl.BlockSpec((D, D), lambda i: (0, 0)),
    ],
    out_specs=pl.BlockSpec((RB, D), lambda i: (i, 0)),
    out_shape=jax.ShapeDtypeStruct((NP, D), jnp.float32),
)


def _tc_last_body(p_ref, y_ref, degp_ref, b_ref, wc_ref, bc_ref,
                  h_ref, o_ref):
    dis = _dis_from_parts(degp_ref[...])
    agg = p_ref[0] + p_ref[1] + y_ref[...]
    h = jnp.maximum(dis[:, None] * agg + b_ref[...], 0.0)
    h_ref[...] = h
    o_ref[...] = jnp.dot(h, wc_ref[...],
                         preferred_element_type=jnp.float32) + bc_ref[...]


_tc_last = pl.pallas_call(
    _tc_last_body,
    grid=(NP // RB,),
    in_specs=[
        pl.BlockSpec((2, RB, D), lambda i: (0, i, 0)),
        pl.BlockSpec((RB, D), lambda i: (i, 0)),
        pl.BlockSpec((NW, RB), lambda i: (0, i)),
        pl.BlockSpec((1, D), lambda i: (0, 0)),
        pl.BlockSpec((D, D), lambda i: (0, 0)),
        pl.BlockSpec((1, D), lambda i: (0, 0)),
    ],
    out_specs=[
        pl.BlockSpec((RB, D), lambda i: (i, 0)),
        pl.BlockSpec((RB, D), lambda i: (i, 0)),
    ],
    out_shape=[
        jax.ShapeDtypeStruct((NP, D), jnp.float32),
        jax.ShapeDtypeStruct((NP, D), jnp.float32),
    ],
)


def kernel(x, edge_index, W1, b1, W2, b2, W3, b3, Wc, bc):
    n_classes = Wc.shape[1]
    ei = edge_index.astype(jnp.int32)
    pad_e = EP - E
    pad_dst = N + (jnp.arange(pad_e, dtype=jnp.int32) % (NP - N))
    src = jnp.concatenate([ei[0], jnp.zeros((pad_e,), jnp.int32)])
    src = src.reshape(NW * NCHUNK, CHUNK)
    dst = jnp.concatenate([ei[1], pad_dst]).reshape(NW * NCHUNK, CHUNK)
    xp = jnp.zeros((NP, D), jnp.float32).at[:N].set(x)
    zeros_row = jnp.zeros((RPT, D), jnp.float32)
    wcp = jnp.zeros((D, D), jnp.float32).at[:, :n_classes].set(Wc)
    bcp = jnp.zeros((1, D), jnp.float32).at[0, :n_classes].set(bc)

    _sc_degree, _sc_scatter = _sc_kernels()
    degp = _sc_degree(dst)
    y1 = _tc_first(degp, xp, W1)
    p1 = _sc_scatter(y1, src, dst, zeros_row).reshape(NC, NP, D)
    y2 = _tc_mid(p1, y1, degp, b1.reshape(1, D), W2)
    p2 = _sc_scatter(y2, src, dst, zeros_row).reshape(NC, NP, D)
    y3 = _tc_mid(p2, y2, degp, b2.reshape(1, D), W3)
    p3 = _sc_scatter(y3, src, dst, zeros_row).reshape(NC, NP, D)
    h3, outp = _tc_last(p3, y3, degp, b3.reshape(1, D), wcp, bcp)
    return (outp[:N, :n_classes], h3[:N])


# speedup vs baseline: 1.3816x; 1.1177x over previous
import functools

import jax
import jax.numpy as jnp
from jax import lax
from jax.experimental import pallas as pl
from jax.experimental.pallas import tpu as pltpu
from jax.experimental.pallas import tpu_sc as plsc

N = 10000
D = 128
NP = 10240
E = 320000
EP = 327680
NC = 2
NS = 16
NW = NC * NS
EPW = EP // NW
CHUNK = 128
NCHUNK = EPW // CHUNK
RPT = NP // NS
RB = 1024
CHUNKS_C0 = 144
CHUNKS_C1 = 16

@functools.cache
def _sc_kernels():
    mesh = plsc.VectorSubcoreMesh(core_axis_name="c", subcore_axis_name="s",
                                  num_cores=NC, num_subcores=NS)

    @functools.partial(
        pl.kernel,
        out_type=jax.ShapeDtypeStruct((NW, NP), jnp.float32),
        mesh=mesh,
        scratch_types=[
            pltpu.VMEM((NCHUNK, CHUNK), jnp.int32),
            pltpu.VMEM((NP,), jnp.float32),
        ],
        compiler_params=pltpu.CompilerParams(needs_layout_passes=False),
    )
    def _sc_degree(dst_hbm, out_hbm, idx_v, hist):
        c = lax.axis_index("c")
        s = lax.axis_index("s")
        wid = s * NC + c

        def zero_body(i, _):
            hist[pl.ds(i * 16, 16)] = jnp.zeros((16,), jnp.float32)
            return 0

        lax.fori_loop(0, NP // 16, zero_body, 0)
        pltpu.sync_copy(dst_hbm.at[pl.ds(wid * NCHUNK, NCHUNK)], idx_v)
        ones = jnp.ones((16,), jnp.float32)

        def hist_body(i, _):
            for j in range(CHUNK // 16):
                idx = idx_v[i, pl.ds(j * 16, 16)]
                plsc.addupdate_scatter(hist, [idx], ones)
            return 0

        lax.fori_loop(0, NCHUNK, hist_body, 0)
        pltpu.sync_copy(hist, out_hbm.at[wid])

    STAGE = 8

    @functools.partial(
        pl.kernel,
        out_type=jax.ShapeDtypeStruct((NC * NP, D), jnp.float32),
        mesh=mesh,
        scratch_types=[
            pltpu.VMEM((STAGE, CHUNK), jnp.int32),
            pltpu.VMEM((STAGE, CHUNK), jnp.int32),
            pltpu.VMEM((CHUNK, D), jnp.float32),
            pltpu.VMEM((CHUNK, D), jnp.float32),
            pltpu.VMEM_SHARED((NP, D), jnp.float32),
            pltpu.SemaphoreType.DMA,
            pltpu.SemaphoreType.DMA,
        ],
    )
    def _sc_scatter(y_hbm, src_hbm, dst_hbm, zeros_hbm, out_hbm,
                    src_v, dst_v, rows_a, rows_b, acc, sem_a, sem_b):
        rows = (rows_a, rows_b)
        sems = (sem_a, sem_b)
        c = lax.axis_index("c")
        s = lax.axis_index("s")
        pltpu.sync_copy(zeros_hbm, acc.at[pl.ds(s * RPT, RPT)])
        plsc.subcore_barrier()

        def run_stage(base):
            pltpu.sync_copy(src_hbm.at[pl.ds(base, STAGE)], src_v)
            pltpu.sync_copy(dst_hbm.at[pl.ds(base, STAGE)], dst_v)
            gdesc = [pltpu.async_copy(y_hbm.at[src_v.at[0]], rows[0],
                                      sems[0]), None]
            for i in range(STAGE):
                b = i % 2
                if i + 1 < STAGE:
                    gdesc[1 - b] = pltpu.async_copy(
                        y_hbm.at[src_v.at[i + 1]], rows[1 - b], sems[1 - b])
                gdesc[b].wait()
                pltpu.sync_copy(rows[b], acc.at[dst_v.at[i]], add=True)

        nst0 = CHUNKS_C0 // STAGE
        nst1 = CHUNKS_C1 // STAGE

        @pl.when(c == 0)
        def _():
            for h in range(nst0):
                run_stage(s * CHUNKS_C0 + h * STAGE)

        @pl.when(c == 1)
        def _():
            for h in range(nst1):
                run_stage(NS * CHUNKS_C0 + s * CHUNKS_C1 + h * STAGE)

        plsc.subcore_barrier()
        pltpu.sync_copy(acc.at[pl.ds(s * RPT, RPT)],
                        out_hbm.at[pl.ds(c * NP + s * RPT, RPT)])

    return _sc_degree, _sc_scatter


def _dis_from_parts(degp):
    deg = jnp.sum(degp, axis=0) + 1.0
    return lax.rsqrt(deg)


def _tc_first_body(degp_ref, x_ref, w_ref, y_ref):
    dis = _dis_from_parts(degp_ref[...])
    xw = jnp.dot(x_ref[...], w_ref[...], preferred_element_type=jnp.float32)
    y_ref[...] = xw * dis[:, None]


_tc_first = pl.pallas_call(
    _tc_first_body,
    grid=(NP // RB,),
    in_specs=[
        pl.BlockSpec((NW, RB), lambda i: (0, i)),
        pl.BlockSpec((RB, D), lambda i: (i, 0)),
        pl.BlockSpec((D, D), lambda i: (0, 0)),
    ],
    out_specs=pl.BlockSpec((RB, D), lambda i: (i, 0)),
    out_shape=jax.ShapeDtypeStruct((NP, D), jnp.float32),
)


def _tc_mid_body(p_ref, y_ref, degp_ref, b_ref, w_ref, o_ref):
    dis = _dis_from_parts(degp_ref[...])
    agg = p_ref[0] + p_ref[1] + y_ref[...]
    h = jnp.maximum(dis[:, None] * agg + b_ref[...], 0.0)
    o_ref[...] = jnp.dot(h, w_ref[...],
                         preferred_element_type=jnp.float32) * dis[:, None]


_tc_mid = pl.pallas_call(
    _tc_mid_body,
    grid=(NP // RB,),
    in_specs=[
        pl.BlockSpec((2, RB, D), lambda i: (0, i, 0)),
        pl.BlockSpec((RB, D), lambda i: (i, 0)),
        pl.BlockSpec((NW, RB), lambda i: (0, i)),
        pl.BlockSpec((1, D), lambda i: (0, 0)),
        pl.BlockSpec((D, D), lambda i: (0, 0)),
    ],
    out_specs=pl.BlockSpec((RB, D), lambda i: (i, 0)),
    out_shape=jax.ShapeDtypeStruct((NP, D), jnp.float32),
)


def _tc_last_body(p_ref, y_ref, degp_ref, b_ref, wc_ref, bc_ref,
                  h_ref, o_ref):
    dis = _dis_from_parts(degp_ref[...])
    agg = p_ref[0] + p_ref[1] + y_ref[...]
    h = jnp.maximum(dis[:, None] * agg + b_ref[...], 0.0)
    h_ref[...] = h
    o_ref[...] = jnp.dot(h, wc_ref[...],
                         preferred_element_type=jnp.float32) + bc_ref[...]


_tc_last = pl.pallas_call(
    _tc_last_body,
    grid=(NP // RB,),
    in_specs=[
        pl.BlockSpec((2, RB, D), lambda i: (0, i, 0)),
        pl.BlockSpec((RB, D), lambda i: (i, 0)),
        pl.BlockSpec((NW, RB), lambda i: (0, i)),
        pl.BlockSpec((1, D), lambda i: (0, 0)),
        pl.BlockSpec((D, D), lambda i: (0, 0)),
        pl.BlockSpec((1, D), lambda i: (0, 0)),
    ],
    out_specs=[
        pl.BlockSpec((RB, D), lambda i: (i, 0)),
        pl.BlockSpec((RB, D), lambda i: (i, 0)),
    ],
    out_shape=[
        jax.ShapeDtypeStruct((NP, D), jnp.float32),
        jax.ShapeDtypeStruct((NP, D), jnp.float32),
    ],
)


def kernel(x, edge_index, W1, b1, W2, b2, W3, b3, Wc, bc):
    n_classes = Wc.shape[1]
    ei = edge_index.astype(jnp.int32)
    pad_e = EP - E
    pad_dst = N + (jnp.arange(pad_e, dtype=jnp.int32) % (NP - N))
    src = jnp.concatenate([ei[0], jnp.zeros((pad_e,), jnp.int32)])
    src = src.reshape(NW * NCHUNK, CHUNK)
    dst = jnp.concatenate([ei[1], pad_dst]).reshape(NW * NCHUNK, CHUNK)
    xp = jnp.zeros((NP, D), jnp.float32).at[:N].set(x)
    zeros_row = jnp.zeros((RPT, D), jnp.float32)
    wcp = jnp.zeros((D, D), jnp.float32).at[:, :n_classes].set(Wc)
    bcp = jnp.zeros((1, D), jnp.float32).at[0, :n_classes].set(bc)

    _sc_degree, _sc_scatter = _sc_kernels()
    degp = _sc_degree(dst)
    y1 = _tc_first(degp, xp, W1)
    p1 = _sc_scatter(y1, src, dst, zeros_row).reshape(NC, NP, D)
    y2 = _tc_mid(p1, y1, degp, b1.reshape(1, D), W2)
    p2 = _sc_scatter(y2, src, dst, zeros_row).reshape(NC, NP, D)
    y3 = _tc_mid(p2, y2, degp, b2.reshape(1, D), W3)
    p3 = _sc_scatter(y3, src, dst, zeros_row).reshape(NC, NP, D)
    h3, outp = _tc_last(p3, y3, degp, b3.reshape(1, D), wcp, bcp)
    return (outp[:N, :n_classes], h3[:N])


# speedup vs baseline: 1.4009x; 1.0140x over previous
import functools

import jax
import jax.numpy as jnp
from jax import lax
from jax.experimental import pallas as pl
from jax.experimental.pallas import tpu as pltpu
from jax.experimental.pallas import tpu_sc as plsc

N = 10000
D = 128
NP = 10240
E = 320000
EP = 327680
NC = 2
NS = 16
NW = NC * NS
EPW = EP // NW
CHUNK = 128
NCHUNK = EPW // CHUNK
RPT = NP // NS
RB = 1024
CHUNKS_C0 = 152
CHUNKS_C1 = 8

@functools.cache
def _sc_kernels():
    mesh = plsc.VectorSubcoreMesh(core_axis_name="c", subcore_axis_name="s",
                                  num_cores=NC, num_subcores=NS)

    @functools.partial(
        pl.kernel,
        out_type=jax.ShapeDtypeStruct((NW, NP), jnp.float32),
        mesh=mesh,
        scratch_types=[
            pltpu.VMEM((NCHUNK, CHUNK), jnp.int32),
            pltpu.VMEM((NP,), jnp.float32),
        ],
        compiler_params=pltpu.CompilerParams(needs_layout_passes=False),
    )
    def _sc_degree(dst_hbm, out_hbm, idx_v, hist):
        c = lax.axis_index("c")
        s = lax.axis_index("s")
        wid = s * NC + c

        def zero_body(i, _):
            hist[pl.ds(i * 16, 16)] = jnp.zeros((16,), jnp.float32)
            return 0

        lax.fori_loop(0, NP // 16, zero_body, 0)
        pltpu.sync_copy(dst_hbm.at[pl.ds(wid * NCHUNK, NCHUNK)], idx_v)
        ones = jnp.ones((16,), jnp.float32)

        def hist_body(i, _):
            for j in range(CHUNK // 16):
                idx = idx_v[i, pl.ds(j * 16, 16)]
                plsc.addupdate_scatter(hist, [idx], ones)
            return 0

        lax.fori_loop(0, NCHUNK, hist_body, 0)
        pltpu.sync_copy(hist, out_hbm.at[wid])

    STAGE = 8

    @functools.partial(
        pl.kernel,
        out_type=jax.ShapeDtypeStruct((NC * NP, D), jnp.float32),
        mesh=mesh,
        scratch_types=[
            pltpu.VMEM((STAGE, CHUNK), jnp.int32),
            pltpu.VMEM((STAGE, CHUNK), jnp.int32),
            pltpu.VMEM((CHUNK, D), jnp.float32),
            pltpu.VMEM((CHUNK, D), jnp.float32),
            pltpu.VMEM_SHARED((NP, D), jnp.float32),
            pltpu.SemaphoreType.DMA,
            pltpu.SemaphoreType.DMA,
        ],
    )
    def _sc_scatter(y_hbm, src_hbm, dst_hbm, zeros_hbm, out_hbm,
                    src_v, dst_v, rows_a, rows_b, acc, sem_a, sem_b):
        rows = (rows_a, rows_b)
        sems = (sem_a, sem_b)
        c = lax.axis_index("c")
        s = lax.axis_index("s")
        pltpu.sync_copy(zeros_hbm, acc.at[pl.ds(s * RPT, RPT)])
        plsc.subcore_barrier()

        def run_stage(base):
            pltpu.sync_copy(src_hbm.at[pl.ds(base, STAGE)], src_v)
            pltpu.sync_copy(dst_hbm.at[pl.ds(base, STAGE)], dst_v)
            gdesc = [pltpu.async_copy(y_hbm.at[src_v.at[0]], rows[0],
                                      sems[0]), None]
            for i in range(STAGE):
                b = i % 2
                if i + 1 < STAGE:
                    gdesc[1 - b] = pltpu.async_copy(
                        y_hbm.at[src_v.at[i + 1]], rows[1 - b], sems[1 - b])
                gdesc[b].wait()
                pltpu.sync_copy(rows[b], acc.at[dst_v.at[i]], add=True)

        nst0 = CHUNKS_C0 // STAGE
        nst1 = CHUNKS_C1 // STAGE

        @pl.when(c == 0)
        def _():
            for h in range(nst0):
                run_stage(s * CHUNKS_C0 + h * STAGE)

        @pl.when(c == 1)
        def _():
            for h in range(nst1):
                run_stage(NS * CHUNKS_C0 + s * CHUNKS_C1 + h * STAGE)

        plsc.subcore_barrier()
        pltpu.sync_copy(acc.at[pl.ds(s * RPT, RPT)],
                        out_hbm.at[pl.ds(c * NP + s * RPT, RPT)])

    return _sc_degree, _sc_scatter


def _dis_from_parts(degp):
    deg = jnp.sum(degp, axis=0) + 1.0
    return lax.rsqrt(deg)


def _tc_first_body(degp_ref, x_ref, w_ref, y_ref):
    dis = _dis_from_parts(degp_ref[...])
    xw = jnp.dot(x_ref[...], w_ref[...], preferred_element_type=jnp.float32)
    y_ref[...] = xw * dis[:, None]


_tc_first = pl.pallas_call(
    _tc_first_body,
    grid=(NP // RB,),
    in_specs=[
        pl.BlockSpec((NW, RB), lambda i: (0, i)),
        pl.BlockSpec((RB, D), lambda i: (i, 0)),
        pl.BlockSpec((D, D), lambda i: (0, 0)),
    ],
    out_specs=pl.BlockSpec((RB, D), lambda i: (i, 0)),
    out_shape=jax.ShapeDtypeStruct((NP, D), jnp.float32),
)


def _tc_mid_body(p_ref, y_ref, degp_ref, b_ref, w_ref, o_ref):
    dis = _dis_from_parts(degp_ref[...])
    agg = p_ref[0] + p_ref[1] + y_ref[...]
    h = jnp.maximum(dis[:, None] * agg + b_ref[...], 0.0)
    o_ref[...] = jnp.dot(h, w_ref[...],
                         preferred_element_type=jnp.float32) * dis[:, None]


_tc_mid = pl.pallas_call(
    _tc_mid_body,
    grid=(NP // RB,),
    in_specs=[
        pl.BlockSpec((2, RB, D), lambda i: (0, i, 0)),
        pl.BlockSpec((RB, D), lambda i: (i, 0)),
        pl.BlockSpec((NW, RB), lambda i: (0, i)),
        pl.BlockSpec((1, D), lambda i: (0, 0)),
        pl.BlockSpec((D, D), lambda i: (0, 0)),
    ],
    out_specs=pl.BlockSpec((RB, D), lambda i: (i, 0)),
    out_shape=jax.ShapeDtypeStruct((NP, D), jnp.float32),
)


def _tc_last_body(p_ref, y_ref, degp_ref, b_ref, wc_ref, bc_ref,
                  h_ref, o_ref):
    dis = _dis_from_parts(degp_ref[...])
    agg = p_ref[0] + p_ref[1] + y_ref[...]
    h = jnp.maximum(dis[:, None] * agg + b_ref[...], 0.0)
    h_ref[...] = h
    o_ref[...] = jnp.dot(h, wc_ref[...],
                         preferred_element_type=jnp.float32) + bc_ref[...]


_tc_last = pl.pallas_call(
    _tc_last_body,
    grid=(NP // RB,),
    in_specs=[
        pl.BlockSpec((2, RB, D), lambda i: (0, i, 0)),
        pl.BlockSpec((RB, D), lambda i: (i, 0)),
        pl.BlockSpec((NW, RB), lambda i: (0, i)),
        pl.BlockSpec((1, D), lambda i: (0, 0)),
        pl.BlockSpec((D, D), lambda i: (0, 0)),
        pl.BlockSpec((1, D), lambda i: (0, 0)),
    ],
    out_specs=[
        pl.BlockSpec((RB, D), lambda i: (i, 0)),
        pl.BlockSpec((RB, D), lambda i: (i, 0)),
    ],
    out_shape=[
        jax.ShapeDtypeStruct((NP, D), jnp.float32),
        jax.ShapeDtypeStruct((NP, D), jnp.float32),
    ],
)


def kernel(x, edge_index, W1, b1, W2, b2, W3, b3, Wc, bc):
    n_classes = Wc.shape[1]
    ei = edge_index.astype(jnp.int32)
    pad_e = EP - E
    pad_dst = N + (jnp.arange(pad_e, dtype=jnp.int32) % (NP - N))
    src = jnp.concatenate([ei[0], jnp.zeros((pad_e,), jnp.int32)])
    src = src.reshape(NW * NCHUNK, CHUNK)
    dst = jnp.concatenate([ei[1], pad_dst]).reshape(NW * NCHUNK, CHUNK)
    xp = jnp.zeros((NP, D), jnp.float32).at[:N].set(x)
    zeros_row = jnp.zeros((RPT, D), jnp.float32)
    wcp = jnp.zeros((D, D), jnp.float32).at[:, :n_classes].set(Wc)
    bcp = jnp.zeros((1, D), jnp.float32).at[0, :n_classes].set(bc)

    _sc_degree, _sc_scatter = _sc_kernels()
    degp = _sc_degree(dst)
    y1 = _tc_first(degp, xp, W1)
    p1 = _sc_scatter(y1, src, dst, zeros_row).reshape(NC, NP, D)
    y2 = _tc_mid(p1, y1, degp, b1.reshape(1, D), W2)
    p2 = _sc_scatter(y2, src, dst, zeros_row).reshape(NC, NP, D)
    y3 = _tc_mid(p2, y2, degp, b2.reshape(1, D), W3)
    p3 = _sc_scatter(y3, src, dst, zeros_row).reshape(NC, NP, D)
    h3, outp = _tc_last(p3, y3, degp, b3.reshape(1, D), wcp, bcp)
    return (outp[:N, :n_classes], h3[:N])
